# Initial kernel scaffold; baseline (speedup 1.0000x reference)
import jax
import jax.numpy as jnp
from jax.experimental import pallas as pl


def kernel(h_drug, h_prot, edge_index, W1_dd, W1_pd, W1_pp, W1_dp, W2_dd, W2_pd, W2_pp, W2_dp):
    raise NotImplementedError("write your pallas kernel here")


# speedup vs baseline: 7.8885x; 7.8885x over previous
import functools

import jax
import jax.numpy as jnp
from jax import lax
from jax.experimental import pallas as pl
from jax.experimental.pallas import tpu as pltpu
from jax.experimental.pallas import tpu_sc as plsc

ND = 10000
NP = 10000
E = 320000
D = 128

NT = 16
C = 80
EPT = E // NT
NCH = EPT // C
GC = 10
NG = NCH // GC
NBUF = 2
RC = C
NRC = ND // RC
KPT = (NRC + NT - 1) // NT


def _sc_dual_segsum(h_drug, h_prot, didx, pidx):
    mesh = plsc.VectorSubcoreMesh(core_axis_name="c", subcore_axis_name="s")

    @functools.partial(
        pl.kernel,
        out_type=(
            jax.ShapeDtypeStruct((ND, D), jnp.float32),
            jax.ShapeDtypeStruct((NP, D), jnp.float32),
        ),
        mesh=mesh,
        scratch_types=[
            pltpu.VMEM_SHARED((ND, D), jnp.float32),
            pltpu.VMEM((2, GC, C), jnp.int32),
            pltpu.VMEM((2, GC, C), jnp.int32),
            pltpu.VMEM((NBUF, C, D), jnp.float32),
            pltpu.SemaphoreType.DMA,
            pltpu.SemaphoreType.DMA((NBUF,)),
        ],
    )
    def k(hd, hp, didx_h, pidx_h, p_out, q_out,
          acc, idx_s, idx_d, rows, sem_i, sem_sc):
        cid = lax.axis_index("c")
        tid = lax.axis_index("s")

        def zero_row(r, carry):
            for cc in range(D // 16):
                rows[0, r, pl.ds(cc * 16, 16)] = jnp.zeros((16,), jnp.float32)
            return carry
        lax.fori_loop(0, RC, zero_row, 0)
        for kk in range(KPT):
            ch = kk * NT + tid

            @pl.when(ch < NRC)
            def _():
                pltpu.sync_copy(rows.at[0], acc.at[pl.ds(ch * RC, RC), :])
        plsc.subcore_barrier()

        def direction(src_tab, sidx_h, dstx_h, out_hbm):
            pltpu.async_copy(sidx_h.at[tid, 0], idx_s.at[0], sem_i)
            pltpu.async_copy(dstx_h.at[tid, 0], idx_d.at[0], sem_i)

            def group(g, carry):
                sg = g % 2
                pltpu.make_async_copy(sidx_h.at[tid, g], idx_s.at[sg],
                                      sem_i).wait()
                pltpu.make_async_copy(dstx_h.at[tid, g], idx_d.at[sg],
                                      sem_i).wait()

                @pl.when(g + 1 < NG)
                def _():
                    pltpu.async_copy(sidx_h.at[tid, g + 1], idx_s.at[1 - sg],
                                     sem_i)
                    pltpu.async_copy(dstx_h.at[tid, g + 1], idx_d.at[1 - sg],
                                     sem_i)

                for j in range(GC):
                    b = j % NBUF
                    if j >= NBUF:
                        pltpu.make_async_copy(
                            rows.at[b], acc.at[idx_d.at[sg, j - NBUF]],
                            sem_sc.at[b]).wait()
                    pltpu.sync_copy(src_tab.at[idx_s.at[sg, j]], rows.at[b])
                    pltpu.async_copy(rows.at[b], acc.at[idx_d.at[sg, j]],
                                     sem_sc.at[b], add=True)
                for j in range(GC - NBUF, GC):
                    b = j % NBUF
                    pltpu.make_async_copy(
                        rows.at[b], acc.at[idx_d.at[sg, j]],
                        sem_sc.at[b]).wait()
                return carry
            lax.fori_loop(0, NG, group, 0)
            plsc.subcore_barrier()

            for kk in range(KPT):
                ch = kk * NT + tid

                @pl.when(ch < NRC)
                def _():
                    b = kk % NBUF
                    pltpu.sync_copy(acc.at[pl.ds(ch * RC, RC), :], rows.at[b])
                    pltpu.sync_copy(rows.at[b], out_hbm.at[pl.ds(ch * RC, RC), :])

        @pl.when(cid == 0)
        def _():
            direction(hp, pidx_h, didx_h, p_out)

        @pl.when(cid == 1)
        def _():
            direction(hd, didx_h, pidx_h, q_out)

    return k(h_drug, h_prot, didx, pidx)


def _tc_dual(hd, ad, hp, ap, w_hd, w_ad, w_hp, w_ap, rd=None, rp=None):
    B = 2000
    G = ND // B
    with_res = rd is not None

    def body(*refs):
        if with_res:
            hd_r, ad_r, hp_r, ap_r, whd, wad, whp, wap, rd_r, rp_r, od, op = refs
        else:
            hd_r, ad_r, hp_r, ap_r, whd, wad, whp, wap, od, op = refs
        accd = (jnp.dot(hd_r[...], whd[...], preferred_element_type=jnp.float32)
                + jnp.dot(ad_r[...], wad[...], preferred_element_type=jnp.float32))
        if with_res:
            accd = accd + rd_r[...]
        od[...] = jnp.maximum(accd, 0.0)
        accp = (jnp.dot(hp_r[...], whp[...], preferred_element_type=jnp.float32)
                + jnp.dot(ap_r[...], wap[...], preferred_element_type=jnp.float32))
        if with_res:
            accp = accp + rp_r[...]
        op[...] = jnp.maximum(accp, 0.0)

    row_spec = pl.BlockSpec((B, D), lambda i: (i, 0))
    w_spec = pl.BlockSpec((D, D), lambda i: (0, 0))
    in_specs = [row_spec] * 4 + [w_spec] * 4 + ([row_spec] * 2 if with_res else [])
    args = (hd, ad, hp, ap, w_hd, w_ad, w_hp, w_ap)
    if with_res:
        args = args + (rd, rp)
    return pl.pallas_call(
        body,
        grid=(G,),
        in_specs=in_specs,
        out_specs=[row_spec, row_spec],
        out_shape=[jax.ShapeDtypeStruct((ND, D), jnp.float32)] * 2,
    )(*args)


def kernel(h_drug, h_prot, edge_index,
           W1_dd, W1_pd, W1_pp, W1_dp,
           W2_dd, W2_pd, W2_pp, W2_dp):
    didx = edge_index[0].reshape(NT, NG, GC, C)
    pidx = edge_index[1].reshape(NT, NG, GC, C)
    p1, q1 = _sc_dual_segsum(h_drug, h_prot, didx, pidx)
    d1, t1 = _tc_dual(h_drug, p1, h_prot, q1, W1_dd, W1_pd, W1_pp, W1_dp)
    p2, q2 = _sc_dual_segsum(d1, t1, didx, pidx)
    out_d, out_p = _tc_dual(d1, p2, t1, q2, W2_dd, W2_pd, W2_pp, W2_dp,
                            rd=h_drug, rp=h_prot)
    return (out_d, out_p)


# speedup vs baseline: 10.8552x; 1.3761x over previous
import functools

import jax
import jax.numpy as jnp
from jax import lax
from jax.experimental import pallas as pl
from jax.experimental.pallas import tpu as pltpu
from jax.experimental.pallas import tpu_sc as plsc

ND = 10000
NP = 10000
E = 320000
D = 128

NT = 16
C = 80
EPT = E // NT
NCH = EPT // C
GC = 10
NG = NCH // GC
NBUF = 4
RC = C
NRC = ND // RC
KPT = (NRC + NT - 1) // NT


def _sc_dual_segsum(h_drug, h_prot, didx, pidx):
    mesh = plsc.VectorSubcoreMesh(core_axis_name="c", subcore_axis_name="s")

    @functools.partial(
        pl.kernel,
        out_type=(
            jax.ShapeDtypeStruct((ND, D), jnp.float32),
            jax.ShapeDtypeStruct((NP, D), jnp.float32),
        ),
        mesh=mesh,
        scratch_types=[
            pltpu.VMEM_SHARED((ND, D), jnp.float32),
            pltpu.VMEM((2, GC, C), jnp.int32),
            pltpu.VMEM((2, GC, C), jnp.int32),
            pltpu.VMEM((NBUF, C, D), jnp.float32),
            pltpu.SemaphoreType.DMA,
            pltpu.SemaphoreType.DMA((NBUF,)),
            pltpu.SemaphoreType.DMA((NBUF,)),
        ],
    )
    def k(hd, hp, didx_h, pidx_h, p_out, q_out,
          acc, idx_s, idx_d, rows, sem_i, sem_g, sem_sc):
        cid = lax.axis_index("c")
        tid = lax.axis_index("s")

        def zero_row(r, carry):
            for cc in range(D // 16):
                rows[0, r, pl.ds(cc * 16, 16)] = jnp.zeros((16,), jnp.float32)
            return carry
        lax.fori_loop(0, RC, zero_row, 0)
        for kk in range(KPT):
            ch = kk * NT + tid

            @pl.when(ch < NRC)
            def _():
                pltpu.sync_copy(rows.at[0], acc.at[pl.ds(ch * RC, RC), :])
        plsc.subcore_barrier()

        def direction(src_tab, sidx_h, dstx_h, out_hbm):
            pltpu.async_copy(sidx_h.at[tid, 0], idx_s.at[0], sem_i)
            pltpu.async_copy(dstx_h.at[tid, 0], idx_d.at[0], sem_i)

            def group(g, carry):
                sg = g % 2
                pltpu.make_async_copy(sidx_h.at[tid, g], idx_s.at[sg],
                                      sem_i).wait()
                pltpu.make_async_copy(dstx_h.at[tid, g], idx_d.at[sg],
                                      sem_i).wait()

                @pl.when(g > 0)
                def _():
                    for b in range(2):
                        pltpu.make_async_copy(
                            rows.at[b], acc.at[idx_d.at[1 - sg, GC - 2 + b]],
                            sem_sc.at[b]).wait()
                for j in range(2):
                    pltpu.async_copy(src_tab.at[idx_s.at[sg, j]],
                                     rows.at[j], sem_g.at[j])

                for j in range(GC):
                    b = j % NBUF
                    if j == 2:
                        @pl.when(g + 1 < NG)
                        def _():
                            pltpu.async_copy(sidx_h.at[tid, g + 1],
                                             idx_s.at[1 - sg], sem_i)
                            pltpu.async_copy(dstx_h.at[tid, g + 1],
                                             idx_d.at[1 - sg], sem_i)
                    if j <= GC - 3:
                        bn = (j + 2) % NBUF
                        if j >= 2:
                            pltpu.make_async_copy(
                                rows.at[bn], acc.at[idx_d.at[sg, j - 2]],
                                sem_sc.at[bn]).wait()
                        else:
                            @pl.when(g > 0)
                            def _(bn=bn, j=j):
                                pltpu.make_async_copy(
                                    rows.at[bn],
                                    acc.at[idx_d.at[1 - sg, GC - 4 + j]],
                                    sem_sc.at[bn]).wait()
                        pltpu.async_copy(src_tab.at[idx_s.at[sg, j + 2]],
                                         rows.at[bn], sem_g.at[bn])
                    pltpu.make_async_copy(src_tab.at[idx_s.at[sg, j]],
                                          rows.at[b], sem_g.at[b]).wait()
                    pltpu.async_copy(rows.at[b], acc.at[idx_d.at[sg, j]],
                                     sem_sc.at[b], add=True)
                return carry
            lax.fori_loop(0, NG, group, 0)

            for j in range(GC - 4, GC):
                pltpu.make_async_copy(
                    rows.at[j % NBUF], acc.at[idx_d.at[(NG - 1) % 2, j]],
                    sem_sc.at[j % NBUF]).wait()
            plsc.subcore_barrier()

            for kk in range(KPT):
                ch = kk * NT + tid

                @pl.when(ch < NRC)
                def _():
                    b = kk % NBUF
                    pltpu.sync_copy(acc.at[pl.ds(ch * RC, RC), :], rows.at[b])
                    pltpu.sync_copy(rows.at[b], out_hbm.at[pl.ds(ch * RC, RC), :])

        @pl.when(cid == 0)
        def _():
            direction(hp, pidx_h, didx_h, p_out)

        @pl.when(cid == 1)
        def _():
            direction(hd, didx_h, pidx_h, q_out)

    return k(h_drug, h_prot, didx, pidx)


def _tc_dual(hd, ad, hp, ap, w_hd, w_ad, w_hp, w_ap, rd=None, rp=None):
    B = 2000
    G = ND // B
    with_res = rd is not None

    def body(*refs):
        if with_res:
            hd_r, ad_r, hp_r, ap_r, whd, wad, whp, wap, rd_r, rp_r, od, op = refs
        else:
            hd_r, ad_r, hp_r, ap_r, whd, wad, whp, wap, od, op = refs
        accd = (jnp.dot(hd_r[...], whd[...], preferred_element_type=jnp.float32)
                + jnp.dot(ad_r[...], wad[...], preferred_element_type=jnp.float32))
        if with_res:
            accd = accd + rd_r[...]
        od[...] = jnp.maximum(accd, 0.0)
        accp = (jnp.dot(hp_r[...], whp[...], preferred_element_type=jnp.float32)
                + jnp.dot(ap_r[...], wap[...], preferred_element_type=jnp.float32))
        if with_res:
            accp = accp + rp_r[...]
        op[...] = jnp.maximum(accp, 0.0)

    row_spec = pl.BlockSpec((B, D), lambda i: (i, 0))
    w_spec = pl.BlockSpec((D, D), lambda i: (0, 0))
    in_specs = [row_spec] * 4 + [w_spec] * 4 + ([row_spec] * 2 if with_res else [])
    args = (hd, ad, hp, ap, w_hd, w_ad, w_hp, w_ap)
    if with_res:
        args = args + (rd, rp)
    return pl.pallas_call(
        body,
        grid=(G,),
        in_specs=in_specs,
        out_specs=[row_spec, row_spec],
        out_shape=[jax.ShapeDtypeStruct((ND, D), jnp.float32)] * 2,
    )(*args)


def kernel(h_drug, h_prot, edge_index,
           W1_dd, W1_pd, W1_pp, W1_dp,
           W2_dd, W2_pd, W2_pp, W2_dp):
    didx = edge_index[0].reshape(NT, NG, GC, C)
    pidx = edge_index[1].reshape(NT, NG, GC, C)
    p1, q1 = _sc_dual_segsum(h_drug, h_prot, didx, pidx)
    d1, t1 = _tc_dual(h_drug, p1, h_prot, q1, W1_dd, W1_pd, W1_pp, W1_dp)
    p2, q2 = _sc_dual_segsum(d1, t1, didx, pidx)
    out_d, out_p = _tc_dual(d1, p2, t1, q2, W2_dd, W2_pd, W2_pp, W2_dp,
                            rd=h_drug, rp=h_prot)
    return (out_d, out_p)


# speedup vs baseline: 10.9518x; 1.0089x over previous
import functools

import jax
import jax.numpy as jnp
from jax import lax
from jax.experimental import pallas as pl
from jax.experimental.pallas import tpu as pltpu
from jax.experimental.pallas import tpu_sc as plsc

ND = 10000
NP = 10000
E = 320000
D = 128

NT = 16
C = 80
EPT = E // NT
NCH = EPT // C
GC = 10
NG = NCH // GC
NBUF = 4
RC = C
NRC = ND // RC
KPT = (NRC + NT - 1) // NT


def _sc_dual_segsum(h_drug, h_prot, didx, pidx):
    mesh = plsc.VectorSubcoreMesh(core_axis_name="c", subcore_axis_name="s")

    @functools.partial(
        pl.kernel,
        out_type=(
            jax.ShapeDtypeStruct((ND, D), jnp.float32),
            jax.ShapeDtypeStruct((NP, D), jnp.float32),
        ),
        mesh=mesh,
        scratch_types=[
            pltpu.VMEM_SHARED((ND, D), jnp.float32),
            pltpu.VMEM((2, GC, C), jnp.int32),
            pltpu.VMEM((2, GC, C), jnp.int32),
            pltpu.VMEM((NBUF, C, D), jnp.float32),
            pltpu.SemaphoreType.DMA,
            pltpu.SemaphoreType.DMA((NBUF,)),
            pltpu.SemaphoreType.DMA((NBUF,)),
        ],
    )
    def k(hd, hp, didx_h, pidx_h, p_out, q_out,
          acc, idx_s, idx_d, rows, sem_i, sem_g, sem_sc):
        cid = lax.axis_index("c")
        tid = lax.axis_index("s")

        def zero_row(r, carry):
            for cc in range(D // 16):
                rows[0, r, pl.ds(cc * 16, 16)] = jnp.zeros((16,), jnp.float32)
            return carry
        lax.fori_loop(0, RC, zero_row, 0)
        for kk in range(KPT):
            ch = kk * NT + tid

            @pl.when(ch < NRC)
            def _():
                pltpu.sync_copy(rows.at[0], acc.at[pl.ds(ch * RC, RC), :])
        plsc.subcore_barrier()

        def direction(src_tab, sidx_h, dstx_h, out_hbm):
            pltpu.async_copy(sidx_h.at[tid, 0], idx_s.at[0], sem_i)
            pltpu.async_copy(dstx_h.at[tid, 0], idx_d.at[0], sem_i)

            def group(g, carry):
                sg = g % 2
                pltpu.make_async_copy(sidx_h.at[tid, g], idx_s.at[sg],
                                      sem_i).wait()
                pltpu.make_async_copy(dstx_h.at[tid, g], idx_d.at[sg],
                                      sem_i).wait()

                @pl.when(g > 0)
                def _():
                    pltpu.make_async_copy(
                        rows.at[1], acc.at[idx_d.at[1 - sg, GC - 1]],
                        sem_sc.at[1]).wait()
                for j in range(3):
                    pltpu.async_copy(src_tab.at[idx_s.at[sg, j]],
                                     rows.at[j], sem_g.at[j])

                for j in range(GC):
                    b = j % NBUF
                    if j == 2:
                        @pl.when(g + 1 < NG)
                        def _():
                            pltpu.async_copy(sidx_h.at[tid, g + 1],
                                             idx_s.at[1 - sg], sem_i)
                            pltpu.async_copy(dstx_h.at[tid, g + 1],
                                             idx_d.at[1 - sg], sem_i)
                    if j >= 1:
                        pltpu.make_async_copy(
                            rows.at[(j - 1) % NBUF],
                            acc.at[idx_d.at[sg, j - 1]],
                            sem_sc.at[(j - 1) % NBUF]).wait()
                    if j <= GC - 4:
                        pltpu.async_copy(src_tab.at[idx_s.at[sg, j + 3]],
                                         rows.at[(j + 3) % NBUF],
                                         sem_g.at[(j + 3) % NBUF])
                    pltpu.make_async_copy(src_tab.at[idx_s.at[sg, j]],
                                          rows.at[b], sem_g.at[b]).wait()
                    pltpu.async_copy(rows.at[b], acc.at[idx_d.at[sg, j]],
                                     sem_sc.at[b], add=True)
                return carry
            lax.fori_loop(0, NG, group, 0)

            pltpu.make_async_copy(
                rows.at[(GC - 1) % NBUF],
                acc.at[idx_d.at[(NG - 1) % 2, GC - 1]],
                sem_sc.at[(GC - 1) % NBUF]).wait()
            plsc.subcore_barrier()

            for kk in range(KPT):
                ch = kk * NT + tid

                @pl.when(ch < NRC)
                def _():
                    b = kk % NBUF
                    pltpu.sync_copy(acc.at[pl.ds(ch * RC, RC), :], rows.at[b])
                    pltpu.sync_copy(rows.at[b], out_hbm.at[pl.ds(ch * RC, RC), :])

        @pl.when(cid == 0)
        def _():
            direction(hp, pidx_h, didx_h, p_out)

        @pl.when(cid == 1)
        def _():
            direction(hd, didx_h, pidx_h, q_out)

    return k(h_drug, h_prot, didx, pidx)


def _tc_dual(hd, ad, hp, ap, w_hd, w_ad, w_hp, w_ap, rd=None, rp=None):
    B = 2000
    G = ND // B
    with_res = rd is not None

    def body(*refs):
        if with_res:
            hd_r, ad_r, hp_r, ap_r, whd, wad, whp, wap, rd_r, rp_r, od, op = refs
        else:
            hd_r, ad_r, hp_r, ap_r, whd, wad, whp, wap, od, op = refs
        accd = (jnp.dot(hd_r[...], whd[...], preferred_element_type=jnp.float32)
                + jnp.dot(ad_r[...], wad[...], preferred_element_type=jnp.float32))
        if with_res:
            accd = accd + rd_r[...]
        od[...] = jnp.maximum(accd, 0.0)
        accp = (jnp.dot(hp_r[...], whp[...], preferred_element_type=jnp.float32)
                + jnp.dot(ap_r[...], wap[...], preferred_element_type=jnp.float32))
        if with_res:
            accp = accp + rp_r[...]
        op[...] = jnp.maximum(accp, 0.0)

    row_spec = pl.BlockSpec((B, D), lambda i: (i, 0))
    w_spec = pl.BlockSpec((D, D), lambda i: (0, 0))
    in_specs = [row_spec] * 4 + [w_spec] * 4 + ([row_spec] * 2 if with_res else [])
    args = (hd, ad, hp, ap, w_hd, w_ad, w_hp, w_ap)
    if with_res:
        args = args + (rd, rp)
    return pl.pallas_call(
        body,
        grid=(G,),
        in_specs=in_specs,
        out_specs=[row_spec, row_spec],
        out_shape=[jax.ShapeDtypeStruct((ND, D), jnp.float32)] * 2,
    )(*args)


def kernel(h_drug, h_prot, edge_index,
           W1_dd, W1_pd, W1_pp, W1_dp,
           W2_dd, W2_pd, W2_pp, W2_dp):
    didx = edge_index[0].reshape(NT, NG, GC, C)
    pidx = edge_index[1].reshape(NT, NG, GC, C)
    p1, q1 = _sc_dual_segsum(h_drug, h_prot, didx, pidx)
    d1, t1 = _tc_dual(h_drug, p1, h_prot, q1, W1_dd, W1_pd, W1_pp, W1_dp)
    p2, q2 = _sc_dual_segsum(d1, t1, didx, pidx)
    out_d, out_p = _tc_dual(d1, p2, t1, q2, W2_dd, W2_pd, W2_pp, W2_dp,
                            rd=h_drug, rp=h_prot)
    return (out_d, out_p)
